# baseline (speedup 1.0000x reference)
import functools

import jax
import jax.numpy as jnp
from jax import lax
from jax.experimental import pallas as pl
from jax.experimental.pallas import tpu as pltpu
from jax.experimental.pallas import tpu_sc as plsc

NC = 2
NS = 16
L = 16
NW = NC * NS
B = 16384
F = 64
NU = 1000000
WIN = 256
NBLK = (NU + WIN - 1) // WIN
WPW = (NBLK + NW - 1) // NW
SLAB = WPW * WIN
FLUSH = 64
SCRAP = 8
SB = B + SCRAP
BPW = B // NW
SWW = 8
SWE = SWW * 256
RING = 3
CMAX = (1000000 + 127) // 128 * 128 - 256
BMAX = 1000000 - SWE

_mesh = plsc.VectorSubcoreMesh(
    core_axis_name="c", subcore_axis_name="s", num_cores=NC, num_subcores=NS
)
_params = pltpu.CompilerParams(
    needs_layout_passes=False, use_tc_tiling_on_sc=True
)


def _stage_body(idx_hbm, tab_hbm, bias_hbm, staged_hbm,
                idx_v, mu, mb, swb, bias_v, win, rowbuf, sidx, wsem, fsem):
    swu = idx_v
    wid = lax.axis_index("s") * NC + lax.axis_index("c")
    lo = wid * SLAB
    hi = jnp.minimum(lo + SLAB, NU)
    nwin = (hi - lo + WIN - 1) // WIN

    pltpu.sync_copy(idx_hbm, idx_v)

    iota = lax.iota(jnp.int32, L)

    for k in range(FLUSH // L):
        sidx[0, pl.ds(k * L, L)] = B + (iota & (SCRAP - 1))

    def comp_body(ch, cnt):
        u = idx_v[pl.ds(ch * L, L)]
        m = (u >= lo) & (u < hi)
        plsc.store_compressed(mu.at[pl.ds(cnt, L)], u, mask=m)
        plsc.store_compressed(mb.at[pl.ds(cnt, L)], ch * L + iota, mask=m)
        return cnt + plsc.all_reduce_population_count(m)[0]

    n = lax.fori_loop(0, B // L, comp_body, jnp.int32(0))
    nch = (n + L - 1) // L

    for pw in range(RING - 1):
        @pl.when(pw < nwin)
        def _():
            pltpu.async_copy(
                tab_hbm.at[:, pl.ds(jnp.minimum(lo + pw * WIN, CMAX), WIN)],
                win.at[pw],
                wsem,
            )

    def win_body(wi, carry):
        cnt_total, ns = carry
        sw_base = lo + (wi // SWW) * SWE
        bs_off = jnp.minimum(sw_base, BMAX)

        def sw_comp(ch, scnt):
            u = mu[pl.ds(ch * L, L)]
            bv = mb[pl.ds(ch * L, L)]
            m = (
                (u >= sw_base)
                & (u < sw_base + SWE)
                & ((ch * L + iota) < n)
            )
            plsc.store_compressed(swu.at[pl.ds(scnt, L)], u, mask=m)
            plsc.store_compressed(swb.at[pl.ds(scnt, L)], bv, mask=m)
            return scnt + plsc.all_reduce_population_count(m)[0]

        def sw_rebuild():
            pltpu.sync_copy(bias_hbm.at[pl.ds(bs_off, SWE)], bias_v)
            return lax.fori_loop(0, nch, sw_comp, jnp.int32(0))

        ns = lax.cond(wi % SWW == 0, sw_rebuild, lambda: ns)

        slot = wi % RING
        c0 = lo + wi * WIN
        c0f = jnp.minimum(c0, CMAX)
        pltpu.make_async_copy(
            tab_hbm.at[:, pl.ds(0, WIN)], win.at[slot], wsem
        ).wait()

        @pl.when(wi + RING - 1 < nwin)
        def _():
            pltpu.async_copy(
                tab_hbm.at[
                    :, pl.ds(jnp.minimum(c0 + (RING - 1) * WIN, CMAX), WIN)
                ],
                win.at[(wi + RING - 1) % RING],
                wsem,
            )

        def scan_body(ch, ct):
            u = swu[pl.ds(ch * L, L)]
            bv = swb[pl.ds(ch * L, L)]
            valid = (ch * L + iota) < ns
            m = (u >= c0) & (u < c0 + WIN) & valid
            cnt2 = plsc.all_reduce_population_count(m)[0]
            rank = plsc.cumsum(m.astype(jnp.int32))

            def ext_body(e, ct2):
                sel = jnp.where(rank == e + 1, m, False)
                us = jnp.sum(jnp.where(sel, u, 0))
                bs = jnp.sum(jnp.where(sel, bv, 0))
                ul = us - c0f
                pos = ct2 % FLUSH
                for kk in range(F // L):
                    g = plsc.load_gather(
                        win.at[slot],
                        [iota + kk * L, jnp.full((L,), ul, jnp.int32)],
                    )
                    rowbuf[pos, pl.ds(kk * L, L)] = g
                bias = plsc.load_gather(
                    bias_v, [jnp.full((L,), us - bs_off, jnp.int32)]
                )
                rowbuf[pos, pl.ds(F, L)] = bias
                plsc.store_scatter(
                    sidx.at[0],
                    [jnp.full((L,), pos, jnp.int32)],
                    jnp.full((L,), bs, jnp.int32),
                    mask=iota == 0,
                )

                @pl.when(pos == FLUSH - 1)
                def _():
                    pltpu.async_copy(
                        rowbuf, staged_hbm.at[sidx.at[0]], fsem
                    ).wait()

                return ct2 + 1

            return lax.fori_loop(0, cnt2, ext_body, ct)

        cnt_total = lax.fori_loop(0, (ns + L - 1) // L, scan_body, cnt_total)
        return (cnt_total, ns)

    lax.fori_loop(0, nwin, win_body, (jnp.int32(0), jnp.int32(0)))

    pltpu.async_copy(rowbuf, staged_hbm.at[sidx.at[0]], fsem).wait()


@functools.partial(
    pl.kernel,
    out_type=(
        jax.ShapeDtypeStruct((SB, 128), jnp.float32),
        jax.ShapeDtypeStruct((SB, 128), jnp.float32),
    ),
    mesh=_mesh,
    scratch_types=[
        pltpu.VMEM((B,), jnp.int32),
        pltpu.VMEM((B,), jnp.int32),
        pltpu.VMEM((B,), jnp.int32),
        pltpu.VMEM((B,), jnp.int32),
        pltpu.VMEM((SWE,), jnp.float32),
        pltpu.VMEM((RING, F, WIN), jnp.float32),
        pltpu.VMEM((FLUSH, 128), jnp.float32),
        pltpu.VMEM((1, FLUSH), jnp.int32),
        pltpu.SemaphoreType.DMA,
        pltpu.SemaphoreType.DMA,
    ],
    compiler_params=_params,
)
def _stage_both(users_hbm, items_hbm, uet_hbm, iet_hbm, ub_hbm, ib_hbm,
                su_hbm, si_hbm, *rest):
    _stage_body(users_hbm, uet_hbm, ub_hbm, su_hbm, *rest)
    _stage_body(items_hbm, iet_hbm, ib_hbm, si_hbm, *rest)


@functools.partial(
    pl.kernel,
    out_type=jax.ShapeDtypeStruct((NW, BPW), jnp.float32),
    mesh=_mesh,
    scratch_types=[
        pltpu.VMEM((WIN, 128), jnp.float32),
        pltpu.VMEM((WIN, 128), jnp.float32),
        pltpu.VMEM((L * L,), jnp.float32),
        pltpu.VMEM((BPW,), jnp.float32),
        pltpu.SemaphoreType.DMA,
    ],
    compiler_params=_params,
)
def _combine(su_hbm, si_hbm, out_hbm, ubuf, ibuf, tbuf, obuf, sem):
    wid = lax.axis_index("s") * NC + lax.axis_index("c")
    base_b = wid * BPW
    iota = lax.iota(jnp.int32, L)
    lane_scaled = iota * L

    def chunk_body(ci, carry):
        sl = pl.ds(base_b + ci * WIN, WIN)
        pltpu.sync_copy(su_hbm.at[sl, :], ubuf)
        pltpu.sync_copy(si_hbm.at[sl, :], ibuf)

        def group_body(g, carry2):
            base = g * L
            for r in range(L):
                row = base + r
                p = ubuf[row, pl.ds(0, L)] * ibuf[row, pl.ds(0, L)]
                for kk in range(1, F // L):
                    p = p + (
                        ubuf[row, pl.ds(kk * L, L)]
                        * ibuf[row, pl.ds(kk * L, L)]
                    )
                plsc.store_scatter(tbuf, [lane_scaled + r], p)
            acc = tbuf[pl.ds(0, L)]
            for l in range(1, L):
                acc = acc + tbuf[pl.ds(l * L, L)]
            rows = base + iota
            col = jnp.full((L,), F, jnp.int32)
            acc = acc + plsc.load_gather(ubuf, [rows, col])
            acc = acc + plsc.load_gather(ibuf, [rows, col])
            obuf[pl.ds(ci * WIN + base, L)] = acc
            return carry2

        return lax.fori_loop(0, WIN // L, group_body, carry)

    lax.fori_loop(0, BPW // WIN, chunk_body, jnp.int32(0))
    pltpu.sync_copy(obuf, out_hbm.at[wid])


def kernel(users, items, user_embeddings, item_embeddings, user_biases, item_biases):
    u = users.astype(jnp.int32)
    it = items.astype(jnp.int32)
    su, si = _stage_both(
        u, it, user_embeddings.T, item_embeddings.T,
        user_biases.reshape(-1), item_biases.reshape(-1),
    )
    out = _combine(su, si)
    return out.reshape(B, 1)


# speedup vs baseline: 1.1069x; 1.1069x over previous
import functools

import jax
import jax.numpy as jnp
from jax import lax
from jax.experimental import pallas as pl
from jax.experimental.pallas import tpu as pltpu
from jax.experimental.pallas import tpu_sc as plsc

NC = 2
NS = 16
L = 16
NW = NC * NS
B = 16384
F = 64
NU = 1000000
WIN = 256
NBLK = (NU + WIN - 1) // WIN
WPW = (NBLK + NW - 1) // NW
SLAB = WPW * WIN
FLUSH = 32
SCRAP = 8
SB = B + SCRAP
BPW = B // NW
SWW = 8
SWE = SWW * 256
RING = 3
CMAX = (1000000 + 127) // 128 * 128 - 256
BMAX = 1000000 - SWE

_mesh = plsc.VectorSubcoreMesh(
    core_axis_name="c", subcore_axis_name="s", num_cores=NC, num_subcores=NS
)
_params = pltpu.CompilerParams(
    needs_layout_passes=False, use_tc_tiling_on_sc=True
)


@functools.partial(
    pl.kernel,
    out_type=jax.ShapeDtypeStruct((SB, 128), jnp.float32),
    mesh=_mesh,
    scratch_types=[
        pltpu.VMEM((B,), jnp.int32),
        pltpu.VMEM((B,), jnp.int32),
        pltpu.VMEM((B,), jnp.int32),
        pltpu.VMEM((B,), jnp.int32),
        pltpu.VMEM((SWE,), jnp.float32),
        pltpu.VMEM((RING, F, WIN), jnp.float32),
        pltpu.VMEM((2, FLUSH, 128), jnp.float32),
        pltpu.VMEM((2, FLUSH), jnp.int32),
        pltpu.SemaphoreType.DMA,
        pltpu.SemaphoreType.DMA,
    ],
    compiler_params=_params,
)
def _stage(idx_hbm, tab_hbm, bias_hbm, staged_hbm,
           idx_v, mu, mb, swb, bias_v, win, rowbuf, sidx, wsem, fsem):
    swu = idx_v
    wid = lax.axis_index("s") * NC + lax.axis_index("c")
    lo = wid * SLAB
    hi = jnp.minimum(lo + SLAB, NU)
    nwin = (hi - lo + WIN - 1) // WIN

    pltpu.sync_copy(idx_hbm, idx_v)

    iota = lax.iota(jnp.int32, L)

    for sl2 in range(2):
        for k in range(FLUSH // L):
            sidx[sl2, pl.ds(k * L, L)] = B + (iota & (SCRAP - 1))

    def comp_body(ch, cnt):
        u = idx_v[pl.ds(ch * L, L)]
        m = (u >= lo) & (u < hi)
        plsc.store_compressed(mu.at[pl.ds(cnt, L)], u, mask=m)
        plsc.store_compressed(mb.at[pl.ds(cnt, L)], ch * L + iota, mask=m)
        return cnt + plsc.all_reduce_population_count(m)[0]

    n = lax.fori_loop(0, B // L, comp_body, jnp.int32(0))
    nch = (n + L - 1) // L

    for pw in range(RING - 1):
        @pl.when(pw < nwin)
        def _():
            pltpu.async_copy(
                tab_hbm.at[:, pl.ds(jnp.minimum(lo + pw * WIN, CMAX), WIN)],
                win.at[pw],
                wsem,
            )

    def win_body(wi, carry):
        cnt_total, ns = carry
        sw_base = lo + (wi // SWW) * SWE
        bs_off = jnp.minimum(sw_base, BMAX)

        def sw_comp(ch, scnt):
            u = mu[pl.ds(ch * L, L)]
            bv = mb[pl.ds(ch * L, L)]
            m = (
                (u >= sw_base)
                & (u < sw_base + SWE)
                & ((ch * L + iota) < n)
            )
            plsc.store_compressed(swu.at[pl.ds(scnt, L)], u, mask=m)
            plsc.store_compressed(swb.at[pl.ds(scnt, L)], bv, mask=m)
            return scnt + plsc.all_reduce_population_count(m)[0]

        def sw_rebuild():
            pltpu.sync_copy(bias_hbm.at[pl.ds(bs_off, SWE)], bias_v)
            return lax.fori_loop(0, nch, sw_comp, jnp.int32(0))

        ns = lax.cond(wi % SWW == 0, sw_rebuild, lambda: ns)

        slot = wi % RING
        c0 = lo + wi * WIN
        c0f = jnp.minimum(c0, CMAX)
        pltpu.make_async_copy(
            tab_hbm.at[:, pl.ds(0, WIN)], win.at[slot], wsem
        ).wait()

        @pl.when(wi + RING - 1 < nwin)
        def _():
            pltpu.async_copy(
                tab_hbm.at[
                    :, pl.ds(jnp.minimum(c0 + (RING - 1) * WIN, CMAX), WIN)
                ],
                win.at[(wi + RING - 1) % RING],
                wsem,
            )

        def scan_body(ch, ct):
            u = swu[pl.ds(ch * L, L)]
            bv = swb[pl.ds(ch * L, L)]
            valid = (ch * L + iota) < ns
            m = (u >= c0) & (u < c0 + WIN) & valid
            cnt2 = plsc.all_reduce_population_count(m)[0]
            rank = plsc.cumsum(m.astype(jnp.int32))

            def ext_body(e, ct2):
                sel = jnp.where(rank == e + 1, m, False)
                us = jnp.sum(jnp.where(sel, u, 0))
                bs = jnp.sum(jnp.where(sel, bv, 0))
                ul = us - c0f
                pos = ct2 % FLUSH
                fslot = (ct2 // FLUSH) % 2
                for kk in range(F // L):
                    g = plsc.load_gather(
                        win.at[slot],
                        [iota + kk * L, jnp.full((L,), ul, jnp.int32)],
                    )
                    rowbuf[fslot, pos, pl.ds(kk * L, L)] = g
                bias = plsc.load_gather(
                    bias_v, [jnp.full((L,), us - bs_off, jnp.int32)]
                )
                rowbuf[fslot, pos, pl.ds(F, L)] = bias
                plsc.store_scatter(
                    sidx.at[fslot],
                    [jnp.full((L,), pos, jnp.int32)],
                    jnp.full((L,), bs, jnp.int32),
                    mask=iota == 0,
                )

                @pl.when(pos == FLUSH - 1)
                def _():
                    @pl.when(ct2 >= 2 * FLUSH - 1)
                    def _():
                        pltpu.make_async_copy(
                            rowbuf.at[0], staged_hbm.at[sidx.at[0]], fsem
                        ).wait()

                    pltpu.async_copy(
                        rowbuf.at[fslot], staged_hbm.at[sidx.at[fslot]], fsem
                    )

                return ct2 + 1

            return lax.fori_loop(0, cnt2, ext_body, ct)

        cnt_total = lax.fori_loop(0, (ns + L - 1) // L, scan_body, cnt_total)
        return (cnt_total, ns)

    ct_fin, _ = lax.fori_loop(0, nwin, win_body, (jnp.int32(0), jnp.int32(0)))

    @pl.when(ct_fin >= FLUSH)
    def _():
        pltpu.make_async_copy(
            rowbuf.at[0], staged_hbm.at[sidx.at[0]], fsem
        ).wait()

    fslot_fin = (ct_fin // FLUSH) % 2
    pltpu.async_copy(
        rowbuf.at[fslot_fin], staged_hbm.at[sidx.at[fslot_fin]], fsem
    ).wait()


@functools.partial(
    pl.kernel,
    out_type=jax.ShapeDtypeStruct((NW, BPW), jnp.float32),
    mesh=_mesh,
    scratch_types=[
        pltpu.VMEM((WIN, 128), jnp.float32),
        pltpu.VMEM((WIN, 128), jnp.float32),
        pltpu.VMEM((L * L,), jnp.float32),
        pltpu.VMEM((BPW,), jnp.float32),
        pltpu.SemaphoreType.DMA,
    ],
    compiler_params=_params,
)
def _combine(su_hbm, si_hbm, out_hbm, ubuf, ibuf, tbuf, obuf, sem):
    wid = lax.axis_index("s") * NC + lax.axis_index("c")
    base_b = wid * BPW
    iota = lax.iota(jnp.int32, L)
    lane_scaled = iota * L

    def chunk_body(ci, carry):
        sl = pl.ds(base_b + ci * WIN, WIN)
        pltpu.sync_copy(su_hbm.at[sl, :], ubuf)
        pltpu.sync_copy(si_hbm.at[sl, :], ibuf)

        def group_body(g, carry2):
            base = g * L
            for r in range(L):
                row = base + r
                p = ubuf[row, pl.ds(0, L)] * ibuf[row, pl.ds(0, L)]
                for kk in range(1, F // L):
                    p = p + (
                        ubuf[row, pl.ds(kk * L, L)]
                        * ibuf[row, pl.ds(kk * L, L)]
                    )
                plsc.store_scatter(tbuf, [lane_scaled + r], p)
            acc = tbuf[pl.ds(0, L)]
            for l in range(1, L):
                acc = acc + tbuf[pl.ds(l * L, L)]
            rows = base + iota
            col = jnp.full((L,), F, jnp.int32)
            acc = acc + plsc.load_gather(ubuf, [rows, col])
            acc = acc + plsc.load_gather(ibuf, [rows, col])
            obuf[pl.ds(ci * WIN + base, L)] = acc
            return carry2

        return lax.fori_loop(0, WIN // L, group_body, carry)

    lax.fori_loop(0, BPW // WIN, chunk_body, jnp.int32(0))
    pltpu.sync_copy(obuf, out_hbm.at[wid])


def kernel(users, items, user_embeddings, item_embeddings, user_biases, item_biases):
    u = users.astype(jnp.int32)
    it = items.astype(jnp.int32)
    su = _stage(u, user_embeddings.T, user_biases.reshape(-1))
    si = _stage(it, item_embeddings.T, item_biases.reshape(-1))
    out = _combine(su, si)
    return out.reshape(B, 1)
